# baseline (speedup 1.0000x reference)
import functools

import jax
import jax.numpy as jnp
from jax import lax
from jax.experimental import pallas as pl
from jax.experimental.pallas import tpu as pltpu
from jax.experimental.pallas import tpu_sc as plsc

K = 2048
G = 16
GS = 128
NC, NS = 2, 16
NW = NC * NS
N_TOK = 16384
BPW = N_TOK // NW
CH = 64
NCHUNK = BPW // CH
L = 16
TB = 8


def _sc_gather_sz(x_flat, sz8, zr8):
    mesh = plsc.VectorSubcoreMesh(
        core_axis_name="c", subcore_axis_name="s",
        num_cores=NC, num_subcores=NS)

    @functools.partial(
        pl.kernel,
        out_type=[
            jax.ShapeDtypeStruct((N_TOK, GS), jnp.float32),
            jax.ShapeDtypeStruct((N_TOK, GS), jnp.int32),
        ],
        mesh=mesh,
        scratch_types=[
            pltpu.VMEM((BPW,), jnp.int32),
            pltpu.VMEM((BPW,), jnp.int32),
            pltpu.VMEM((2, CH, GS), jnp.float32),
            pltpu.VMEM((2, CH, GS), jnp.int32),
            pltpu.SemaphoreType.DMA,
            pltpu.SemaphoreType.DMA,
        ],
    )
    def k(x_hbm, s8_hbm, z8_hbm, sgg_hbm, zgg_hbm,
          idx_v, idx8_v, sgg_v, zgg_v, ssem0, ssem1):
        wid = lax.axis_index("s") * NC + lax.axis_index("c")
        base = wid * BPW
        pltpu.sync_copy(x_hbm.at[pl.ds(base, BPW)], idx_v)
        for i in range(BPW // L):
            idx8_v[pl.ds(i * L, L)] = jax.lax.shift_right_logical(
                idx_v[pl.ds(i * L, L)], 3)

        ssems = (ssem0, ssem1)

        def start(c):
            b = c % 2
            return (
                pltpu.async_copy(
                    s8_hbm.at[idx8_v.at[pl.ds(c * CH, CH)]],
                    sgg_v.at[b], ssems[b]),
                pltpu.async_copy(
                    z8_hbm.at[idx8_v.at[pl.ds(c * CH, CH)]],
                    zgg_v.at[b], ssems[b]),
            )

        cps = start(0)
        for c in range(NCHUNK):
            nxt = start(c + 1) if c + 1 < NCHUNK else None
            b = c % 2
            for cp in cps:
                cp.wait()
            pltpu.sync_copy(sgg_v.at[b], sgg_hbm.at[pl.ds(base + c * CH, CH)])
            pltpu.sync_copy(zgg_v.at[b], zgg_hbm.at[pl.ds(base + c * CH, CH)])
            cps = nxt

    return k(x_flat, sz8, zr8)


def _dq_body(idx_ref, *refs):
    qrefs = refs[:TB]
    sgref, zgref, ohref, eref, oref = refs[TB:]
    q = jnp.concatenate([r[...] for r in qrefs], axis=0).astype(jnp.float32)
    sgg = sgref[...]
    zgg = zgref[...].astype(jnp.float32)
    oh = ohref[...]
    s16 = jnp.zeros((TB, G), jnp.float32)
    z16 = jnp.zeros((TB, G), jnp.float32)
    for j in range(8):
        m = oh[:, j:j + 1]
        s16 = s16 + m * sgg[:, j * G:(j + 1) * G]
        z16 = z16 + m * zgg[:, j * G:(j + 1) * G]
    e = eref[...]
    sw = jnp.dot(s16, e, preferred_element_type=jnp.float32)
    zw = jnp.dot(z16, e, preferred_element_type=jnp.float32)
    oref[...] = (q - zw) * sw


def _tc_dequant(x_flat, qweights, sgg, zgg, oh8, emat):
    qspecs = [
        pl.BlockSpec((None, 1, K), functools.partial(
            lambda j, i, idx_ref: (idx_ref[i * TB + j], 0, 0), j))
        for j in range(TB)
    ]
    grid_spec = pltpu.PrefetchScalarGridSpec(
        num_scalar_prefetch=1,
        grid=(N_TOK // TB,),
        in_specs=[
            *qspecs,
            pl.BlockSpec((TB, GS), lambda i, idx_ref: (i, 0)),
            pl.BlockSpec((TB, GS), lambda i, idx_ref: (i, 0)),
            pl.BlockSpec((TB, 8), lambda i, idx_ref: (i, 0)),
            pl.BlockSpec((G, K), lambda i, idx_ref: (0, 0)),
        ],
        out_specs=pl.BlockSpec((TB, K), lambda i, idx_ref: (i, 0)),
    )
    qw3 = qweights.reshape(qweights.shape[0], 1, K)
    return pl.pallas_call(
        _dq_body,
        grid_spec=grid_spec,
        out_shape=jax.ShapeDtypeStruct((N_TOK, K), jnp.float32),
    )(x_flat, *([qw3] * TB), sgg, zgg, oh8, emat)


def kernel(x, qweights, scales, zeros):
    shape = x.shape
    x_flat = x.reshape(-1)
    sz8 = scales.reshape(-1, GS)
    zr8 = zeros.reshape(-1, GS)
    sgg, zgg = _sc_gather_sz(x_flat, sz8, zr8)
    oh8 = (jnp.mod(x_flat, 8)[:, None]
           == jnp.arange(8, dtype=jnp.int32)[None, :]).astype(jnp.float32)
    emat = (jnp.arange(K, dtype=jnp.int32)[None, :] // GS
            == jnp.arange(G, dtype=jnp.int32)[:, None]).astype(jnp.float32)
    out = _tc_dequant(x_flat, qweights, sgg, zgg, oh8, emat)
    return out.reshape(*shape, K)


# speedup vs baseline: 1.2617x; 1.2617x over previous
import functools

import jax
import jax.numpy as jnp
from jax import lax
from jax.experimental import pallas as pl
from jax.experimental.pallas import tpu as pltpu
from jax.experimental.pallas import tpu_sc as plsc

K = 2048
G = 16
GS = 128
NC, NS = 2, 16
NW = NC * NS
N_TOK = 16384
BPW = N_TOK // NW
CH = 64
NCHUNK = BPW // CH
L = 16
TB = 8
RB = 32


def _sc_gather_sz(x_flat, sz8, zr8):
    mesh = plsc.VectorSubcoreMesh(
        core_axis_name="c", subcore_axis_name="s",
        num_cores=NC, num_subcores=NS)

    @functools.partial(
        pl.kernel,
        out_type=[
            jax.ShapeDtypeStruct((N_TOK, GS), jnp.float32),
            jax.ShapeDtypeStruct((N_TOK, GS), jnp.int32),
        ],
        mesh=mesh,
        scratch_types=[
            pltpu.VMEM((BPW,), jnp.int32),
            pltpu.VMEM((BPW,), jnp.int32),
            pltpu.VMEM((2, CH, GS), jnp.float32),
            pltpu.VMEM((2, CH, GS), jnp.int32),
            pltpu.SemaphoreType.DMA,
            pltpu.SemaphoreType.DMA,
        ],
    )
    def k(x_hbm, s8_hbm, z8_hbm, sgg_hbm, zgg_hbm,
          idx_v, idx8_v, sgg_v, zgg_v, ssem0, ssem1):
        wid = lax.axis_index("s") * NC + lax.axis_index("c")
        base = wid * BPW
        pltpu.sync_copy(x_hbm.at[pl.ds(base, BPW)], idx_v)
        for i in range(BPW // L):
            idx8_v[pl.ds(i * L, L)] = jax.lax.shift_right_logical(
                idx_v[pl.ds(i * L, L)], 3)

        ssems = (ssem0, ssem1)

        def start(c):
            b = c % 2
            return (
                pltpu.async_copy(
                    s8_hbm.at[idx8_v.at[pl.ds(c * CH, CH)]],
                    sgg_v.at[b], ssems[b]),
                pltpu.async_copy(
                    z8_hbm.at[idx8_v.at[pl.ds(c * CH, CH)]],
                    zgg_v.at[b], ssems[b]),
            )

        cps = start(0)
        for c in range(NCHUNK):
            nxt = start(c + 1) if c + 1 < NCHUNK else None
            b = c % 2
            for cp in cps:
                cp.wait()
            pltpu.sync_copy(sgg_v.at[b], sgg_hbm.at[pl.ds(base + c * CH, CH)])
            pltpu.sync_copy(zgg_v.at[b], zgg_hbm.at[pl.ds(base + c * CH, CH)])
            cps = nxt

    return k(x_flat, sz8, zr8)


def _dq_body(idx_ref, *refs):
    qrefs = refs[:TB]
    sgref, zgref, ohref, oh32ref, eref, oref = refs[TB:]
    qbig = jnp.concatenate([r[...] for r in qrefs], axis=0)
    sel = oh32ref[...]
    q = jax.lax.dot_general(
        sel, qbig, (((1,), (0,)), ((), ())),
        preferred_element_type=jnp.int32).astype(jnp.float32)

    sgg = sgref[...]
    zgg = zgref[...].astype(jnp.float32)
    oh = ohref[...]
    s16 = jnp.zeros((TB, G), jnp.float32)
    z16 = jnp.zeros((TB, G), jnp.float32)
    for j in range(8):
        m = oh[:, j:j + 1]
        s16 = s16 + m * sgg[:, j * G:(j + 1) * G]
        z16 = z16 + m * zgg[:, j * G:(j + 1) * G]
    e = eref[...]
    sw = jax.lax.dot_general(
        s16, e, (((1,), (0,)), ((), ())),
        preferred_element_type=jnp.float32,
        precision=jax.lax.Precision.HIGHEST)
    zw = jax.lax.dot_general(
        z16, e, (((1,), (0,)), ((), ())),
        preferred_element_type=jnp.float32,
        precision=jax.lax.Precision.HIGHEST)
    oref[...] = ((q - zw) * sw).reshape(1, TB, K)


NSTEP = N_TOK // TB


def _tc_dequant(xs, qweights, sgg, zgg, oh8, oh32, emat):
    qspecs = [
        pl.BlockSpec((RB, K), functools.partial(
            lambda j, i, idx_ref: (idx_ref[j * NSTEP + i] // RB, 0), j))
        for j in range(TB)
    ]
    grid_spec = pltpu.PrefetchScalarGridSpec(
        num_scalar_prefetch=1,
        grid=(NSTEP,),
        in_specs=[
            *qspecs,
            pl.BlockSpec((TB, GS), lambda i, idx_ref: (i, 0)),
            pl.BlockSpec((TB, GS), lambda i, idx_ref: (i, 0)),
            pl.BlockSpec((TB, 8), lambda i, idx_ref: (i, 0)),
            pl.BlockSpec((None, TB, TB * RB), lambda i, idx_ref: (i, 0, 0)),
            pl.BlockSpec((G, K), lambda i, idx_ref: (0, 0)),
        ],
        out_specs=pl.BlockSpec((1, TB, K), lambda i, idx_ref: (i, 0, 0)),
    )
    return pl.pallas_call(
        _dq_body,
        grid_spec=grid_spec,
        out_shape=jax.ShapeDtypeStruct((NSTEP, TB, K), jnp.float32),
    )(xs, *([qweights] * TB), sgg, zgg, oh8, oh32, emat)


def _sc_unpermute(outp2, ip):
    RCH = 16
    mesh = plsc.VectorSubcoreMesh(
        core_axis_name="c", subcore_axis_name="s",
        num_cores=NC, num_subcores=NS)

    @functools.partial(
        pl.kernel,
        out_type=jax.ShapeDtypeStruct((N_TOK, K), jnp.float32),
        mesh=mesh,
        scratch_types=[
            pltpu.VMEM((BPW,), jnp.int32),
            pltpu.VMEM((2, RCH, K), jnp.float32),
            pltpu.SemaphoreType.DMA,
            pltpu.SemaphoreType.DMA,
        ],
    )
    def k(ip_hbm, src_hbm, out_hbm, ipx_v, buf_v, sem0, sem1):
        wid = lax.axis_index("s") * NC + lax.axis_index("c")
        base = wid * BPW
        pltpu.sync_copy(ip_hbm.at[pl.ds(base, BPW)], ipx_v)
        sems = (sem0, sem1)

        def start(c):
            b = c % 2
            return pltpu.async_copy(
                src_hbm.at[ipx_v.at[pl.ds(c * RCH, RCH)]],
                buf_v.at[b], sems[b])

        cp = start(0)
        for c in range(BPW // RCH):
            nxt = start(c + 1) if c + 1 < BPW // RCH else None
            cp.wait()
            pltpu.sync_copy(buf_v.at[c % 2],
                            out_hbm.at[pl.ds(base + c * RCH, RCH)])
            cp = nxt

    return k(ip, outp2)


def kernel(x, qweights, scales, zeros):
    shape = x.shape
    x_flat = x.reshape(-1)
    sz8 = scales.reshape(-1, GS)
    zr8 = zeros.reshape(-1, GS)
    iota_t = jnp.arange(N_TOK, dtype=jnp.int32)
    perm = jnp.argsort(x_flat)
    xs = jnp.take(x_flat, perm)
    inv = jnp.zeros((N_TOK,), jnp.int32).at[perm].set(iota_t)
    ip = (inv % NSTEP) * TB + inv // NSTEP
    posr = (iota_t % TB) * NSTEP + iota_t // TB
    xsp = jnp.take(xs, posr)
    sgg, zgg = _sc_gather_sz(xsp, sz8, zr8)
    oh8 = (jnp.mod(xsp, 8)[:, None]
           == jnp.arange(8, dtype=jnp.int32)[None, :]).astype(jnp.float32)
    sel_col = (iota_t % TB) * RB + jnp.mod(xsp, RB)
    oh32 = (sel_col[:, None] == jnp.arange(TB * RB, dtype=jnp.int32)[None, :]
            ).astype(jnp.int8).reshape(NSTEP, TB, TB * RB)
    emat = (jnp.arange(K, dtype=jnp.int32)[None, :] // GS
            == jnp.arange(G, dtype=jnp.int32)[:, None]).astype(jnp.float32)
    outp = _tc_dequant(xs, qweights, sgg, zgg, oh8, oh32, emat)
    out = _sc_unpermute(outp.reshape(N_TOK, K), ip)
    return out.reshape(*shape, K)


# speedup vs baseline: 1.5754x; 1.2486x over previous
import functools

import jax
import jax.numpy as jnp
from jax import lax
from jax.experimental import pallas as pl
from jax.experimental.pallas import tpu as pltpu
from jax.experimental.pallas import tpu_sc as plsc

K = 2048
G = 16
GS = 128
NC, NS = 2, 16
NW = NC * NS
N_TOK = 16384
BPW = N_TOK // NW
CH = 64
NCHUNK = BPW // CH
L = 16
TB = 16
RB = 32


def _sc_gather_sz(x_flat, sz8, zr8):
    mesh = plsc.VectorSubcoreMesh(
        core_axis_name="c", subcore_axis_name="s",
        num_cores=NC, num_subcores=NS)

    @functools.partial(
        pl.kernel,
        out_type=[
            jax.ShapeDtypeStruct((N_TOK, GS), jnp.float32),
            jax.ShapeDtypeStruct((N_TOK, GS), jnp.int32),
        ],
        mesh=mesh,
        scratch_types=[
            pltpu.VMEM((BPW,), jnp.int32),
            pltpu.VMEM((BPW,), jnp.int32),
            pltpu.VMEM((2, CH, GS), jnp.float32),
            pltpu.VMEM((2, CH, GS), jnp.int32),
            pltpu.SemaphoreType.DMA,
            pltpu.SemaphoreType.DMA,
        ],
    )
    def k(x_hbm, s8_hbm, z8_hbm, sgg_hbm, zgg_hbm,
          idx_v, idx8_v, sgg_v, zgg_v, ssem0, ssem1):
        wid = lax.axis_index("s") * NC + lax.axis_index("c")
        base = wid * BPW
        pltpu.sync_copy(x_hbm.at[pl.ds(base, BPW)], idx_v)
        for i in range(BPW // L):
            idx8_v[pl.ds(i * L, L)] = jax.lax.shift_right_logical(
                idx_v[pl.ds(i * L, L)], 3)

        ssems = (ssem0, ssem1)

        def start(c):
            b = c % 2
            return (
                pltpu.async_copy(
                    s8_hbm.at[idx8_v.at[pl.ds(c * CH, CH)]],
                    sgg_v.at[b], ssems[b]),
                pltpu.async_copy(
                    z8_hbm.at[idx8_v.at[pl.ds(c * CH, CH)]],
                    zgg_v.at[b], ssems[b]),
            )

        cps = start(0)
        for c in range(NCHUNK):
            nxt = start(c + 1) if c + 1 < NCHUNK else None
            b = c % 2
            for cp in cps:
                cp.wait()
            pltpu.sync_copy(sgg_v.at[b], sgg_hbm.at[pl.ds(base + c * CH, CH)])
            pltpu.sync_copy(zgg_v.at[b], zgg_hbm.at[pl.ds(base + c * CH, CH)])
            cps = nxt

    return k(x_flat, sz8, zr8)


def _dq_body(idx_ref, *refs):
    qrefs = refs[:TB]
    sgref, zgref, ohref, oh32ref, eref, oref = refs[TB:]
    qbig = jnp.concatenate([r[...] for r in qrefs], axis=0)
    sel = oh32ref[...]
    q = jax.lax.dot_general(
        sel, qbig, (((1,), (0,)), ((), ())),
        preferred_element_type=jnp.int32).astype(jnp.float32)

    sgg = sgref[...]
    zgg = zgref[...].astype(jnp.float32)
    oh = ohref[...]
    s16 = jnp.zeros((TB, G), jnp.float32)
    z16 = jnp.zeros((TB, G), jnp.float32)
    for j in range(8):
        m = oh[:, j:j + 1]
        s16 = s16 + m * sgg[:, j * G:(j + 1) * G]
        z16 = z16 + m * zgg[:, j * G:(j + 1) * G]
    e = eref[...]
    sw = jax.lax.dot_general(
        s16, e, (((1,), (0,)), ((), ())),
        preferred_element_type=jnp.float32,
        precision=jax.lax.Precision.HIGHEST)
    zw = jax.lax.dot_general(
        z16, e, (((1,), (0,)), ((), ())),
        preferred_element_type=jnp.float32,
        precision=jax.lax.Precision.HIGHEST)
    oref[...] = ((q - zw) * sw).reshape(1, TB, K)


NSTEP = N_TOK // TB


def _tc_dequant(xs, qweights, sgg, zgg, oh8, oh32, emat):
    qspecs = [
        pl.BlockSpec((RB, K), functools.partial(
            lambda j, i, idx_ref: (idx_ref[j * NSTEP + i] // RB, 0), j))
        for j in range(TB)
    ]
    grid_spec = pltpu.PrefetchScalarGridSpec(
        num_scalar_prefetch=1,
        grid=(NSTEP,),
        in_specs=[
            *qspecs,
            pl.BlockSpec((TB, GS), lambda i, idx_ref: (i, 0)),
            pl.BlockSpec((TB, GS), lambda i, idx_ref: (i, 0)),
            pl.BlockSpec((TB, 8), lambda i, idx_ref: (i, 0)),
            pl.BlockSpec((None, TB, TB * RB), lambda i, idx_ref: (i, 0, 0)),
            pl.BlockSpec((G, K), lambda i, idx_ref: (0, 0)),
        ],
        out_specs=pl.BlockSpec((1, TB, K), lambda i, idx_ref: (i, 0, 0)),
    )
    return pl.pallas_call(
        _dq_body,
        grid_spec=grid_spec,
        out_shape=jax.ShapeDtypeStruct((NSTEP, TB, K), jnp.float32),
    )(xs, *([qweights] * TB), sgg, zgg, oh8, oh32, emat)


def _sc_unpermute(outp2, ip):
    RCH = 16
    mesh = plsc.VectorSubcoreMesh(
        core_axis_name="c", subcore_axis_name="s",
        num_cores=NC, num_subcores=NS)

    @functools.partial(
        pl.kernel,
        out_type=jax.ShapeDtypeStruct((N_TOK, K), jnp.float32),
        mesh=mesh,
        scratch_types=[
            pltpu.VMEM((BPW,), jnp.int32),
            pltpu.VMEM((2, RCH, K), jnp.float32),
            pltpu.SemaphoreType.DMA,
            pltpu.SemaphoreType.DMA,
        ],
    )
    def k(ip_hbm, src_hbm, out_hbm, ipx_v, buf_v, sem0, sem1):
        wid = lax.axis_index("s") * NC + lax.axis_index("c")
        base = wid * BPW
        pltpu.sync_copy(ip_hbm.at[pl.ds(base, BPW)], ipx_v)
        sems = (sem0, sem1)

        def start(c):
            b = c % 2
            return pltpu.async_copy(
                src_hbm.at[ipx_v.at[pl.ds(c * RCH, RCH)]],
                buf_v.at[b], sems[b])

        cp = start(0)
        for c in range(BPW // RCH):
            nxt = start(c + 1) if c + 1 < BPW // RCH else None
            cp.wait()
            pltpu.sync_copy(buf_v.at[c % 2],
                            out_hbm.at[pl.ds(base + c * RCH, RCH)])
            cp = nxt

    return k(ip, outp2)


def kernel(x, qweights, scales, zeros):
    shape = x.shape
    x_flat = x.reshape(-1)
    sz8 = scales.reshape(-1, GS)
    zr8 = zeros.reshape(-1, GS)
    iota_t = jnp.arange(N_TOK, dtype=jnp.int32)
    perm = jnp.argsort(x_flat)
    xs = jnp.take(x_flat, perm)
    inv = jnp.zeros((N_TOK,), jnp.int32).at[perm].set(iota_t)
    ip = (inv % NSTEP) * TB + inv // NSTEP
    posr = (iota_t % TB) * NSTEP + iota_t // TB
    xsp = jnp.take(xs, posr)
    sgg, zgg = _sc_gather_sz(xsp, sz8, zr8)
    oh8 = (jnp.mod(xsp, 8)[:, None]
           == jnp.arange(8, dtype=jnp.int32)[None, :]).astype(jnp.float32)
    sel_col = (iota_t % TB) * RB + jnp.mod(xsp, RB)
    oh32 = (sel_col[:, None] == jnp.arange(TB * RB, dtype=jnp.int32)[None, :]
            ).astype(jnp.int8).reshape(NSTEP, TB, TB * RB)
    emat = (jnp.arange(K, dtype=jnp.int32)[None, :] // GS
            == jnp.arange(G, dtype=jnp.int32)[:, None]).astype(jnp.float32)
    outp = _tc_dequant(xs, qweights, sgg, zgg, oh8, oh32, emat)
    out = _sc_unpermute(outp.reshape(N_TOK, K), ip)
    return out.reshape(*shape, K)


# speedup vs baseline: 1.6064x; 1.0197x over previous
import functools

import jax
import jax.numpy as jnp
from jax import lax
from jax.experimental import pallas as pl
from jax.experimental.pallas import tpu as pltpu
from jax.experimental.pallas import tpu_sc as plsc

K = 2048
G = 16
GS = 128
NC, NS = 2, 16
NW = NC * NS
N_TOK = 16384
BPW = N_TOK // NW
CH = 64
NCHUNK = BPW // CH
L = 16
TB = 32
RB = 32


def _sc_gather_sz(x_flat, sz8, zr8):
    mesh = plsc.VectorSubcoreMesh(
        core_axis_name="c", subcore_axis_name="s",
        num_cores=NC, num_subcores=NS)

    @functools.partial(
        pl.kernel,
        out_type=[
            jax.ShapeDtypeStruct((N_TOK, GS), jnp.float32),
            jax.ShapeDtypeStruct((N_TOK, GS), jnp.int32),
        ],
        mesh=mesh,
        scratch_types=[
            pltpu.VMEM((BPW,), jnp.int32),
            pltpu.VMEM((BPW,), jnp.int32),
            pltpu.VMEM((2, CH, GS), jnp.float32),
            pltpu.VMEM((2, CH, GS), jnp.int32),
            pltpu.SemaphoreType.DMA,
            pltpu.SemaphoreType.DMA,
        ],
    )
    def k(x_hbm, s8_hbm, z8_hbm, sgg_hbm, zgg_hbm,
          idx_v, idx8_v, sgg_v, zgg_v, ssem0, ssem1):
        wid = lax.axis_index("s") * NC + lax.axis_index("c")
        base = wid * BPW
        pltpu.sync_copy(x_hbm.at[pl.ds(base, BPW)], idx_v)
        for i in range(BPW // L):
            idx8_v[pl.ds(i * L, L)] = jax.lax.shift_right_logical(
                idx_v[pl.ds(i * L, L)], 3)

        ssems = (ssem0, ssem1)

        def start(c):
            b = c % 2
            return (
                pltpu.async_copy(
                    s8_hbm.at[idx8_v.at[pl.ds(c * CH, CH)]],
                    sgg_v.at[b], ssems[b]),
                pltpu.async_copy(
                    z8_hbm.at[idx8_v.at[pl.ds(c * CH, CH)]],
                    zgg_v.at[b], ssems[b]),
            )

        cps = start(0)
        for c in range(NCHUNK):
            nxt = start(c + 1) if c + 1 < NCHUNK else None
            b = c % 2
            for cp in cps:
                cp.wait()
            pltpu.sync_copy(sgg_v.at[b], sgg_hbm.at[pl.ds(base + c * CH, CH)])
            pltpu.sync_copy(zgg_v.at[b], zgg_hbm.at[pl.ds(base + c * CH, CH)])
            cps = nxt

    return k(x_flat, sz8, zr8)


def _dq_body(idx_ref, *refs):
    qrefs = refs[:TB]
    sgref, zgref, ohref, oh32ref, eref, oref = refs[TB:]
    qbig = jnp.concatenate([r[...] for r in qrefs], axis=0)
    sel = oh32ref[...]
    q = jax.lax.dot_general(
        sel, qbig, (((1,), (0,)), ((), ())),
        preferred_element_type=jnp.int32).astype(jnp.float32)

    sgg = sgref[...]
    zgg = zgref[...].astype(jnp.float32)
    oh = ohref[...]
    s16 = jnp.zeros((TB, G), jnp.float32)
    z16 = jnp.zeros((TB, G), jnp.float32)
    for j in range(8):
        m = oh[:, j:j + 1]
        s16 = s16 + m * sgg[:, j * G:(j + 1) * G]
        z16 = z16 + m * zgg[:, j * G:(j + 1) * G]
    e = eref[...]
    sw = jax.lax.dot_general(
        s16, e, (((1,), (0,)), ((), ())),
        preferred_element_type=jnp.float32,
        precision=jax.lax.Precision.HIGHEST)
    zw = jax.lax.dot_general(
        z16, e, (((1,), (0,)), ((), ())),
        preferred_element_type=jnp.float32,
        precision=jax.lax.Precision.HIGHEST)
    oref[...] = ((q - zw) * sw).reshape(1, TB, K)


NSTEP = N_TOK // TB


def _tc_dequant(xs, qweights, sgg, zgg, oh8, oh32, emat):
    qspecs = [
        pl.BlockSpec((RB, K), functools.partial(
            lambda j, i, idx_ref: (idx_ref[j * NSTEP + i] // RB, 0), j))
        for j in range(TB)
    ]
    grid_spec = pltpu.PrefetchScalarGridSpec(
        num_scalar_prefetch=1,
        grid=(NSTEP,),
        in_specs=[
            *qspecs,
            pl.BlockSpec((TB, GS), lambda i, idx_ref: (i, 0)),
            pl.BlockSpec((TB, GS), lambda i, idx_ref: (i, 0)),
            pl.BlockSpec((TB, 8), lambda i, idx_ref: (i, 0)),
            pl.BlockSpec((None, TB, TB * RB), lambda i, idx_ref: (i, 0, 0)),
            pl.BlockSpec((G, K), lambda i, idx_ref: (0, 0)),
        ],
        out_specs=pl.BlockSpec((1, TB, K), lambda i, idx_ref: (i, 0, 0)),
    )
    return pl.pallas_call(
        _dq_body,
        grid_spec=grid_spec,
        out_shape=jax.ShapeDtypeStruct((NSTEP, TB, K), jnp.float32),
    )(xs, *([qweights] * TB), sgg, zgg, oh8, oh32, emat)


def _sc_unpermute(outp2, ip):
    RCH = 16
    mesh = plsc.VectorSubcoreMesh(
        core_axis_name="c", subcore_axis_name="s",
        num_cores=NC, num_subcores=NS)

    @functools.partial(
        pl.kernel,
        out_type=jax.ShapeDtypeStruct((N_TOK, K), jnp.float32),
        mesh=mesh,
        scratch_types=[
            pltpu.VMEM((BPW,), jnp.int32),
            pltpu.VMEM((2, RCH, K), jnp.float32),
            pltpu.SemaphoreType.DMA,
            pltpu.SemaphoreType.DMA,
        ],
    )
    def k(ip_hbm, src_hbm, out_hbm, ipx_v, buf_v, sem0, sem1):
        wid = lax.axis_index("s") * NC + lax.axis_index("c")
        base = wid * BPW
        pltpu.sync_copy(ip_hbm.at[pl.ds(base, BPW)], ipx_v)
        sems = (sem0, sem1)

        def start(c):
            b = c % 2
            return pltpu.async_copy(
                src_hbm.at[ipx_v.at[pl.ds(c * RCH, RCH)]],
                buf_v.at[b], sems[b])

        cp = start(0)
        for c in range(BPW // RCH):
            nxt = start(c + 1) if c + 1 < BPW // RCH else None
            cp.wait()
            pltpu.sync_copy(buf_v.at[c % 2],
                            out_hbm.at[pl.ds(base + c * RCH, RCH)])
            cp = nxt

    return k(ip, outp2)


def kernel(x, qweights, scales, zeros):
    shape = x.shape
    x_flat = x.reshape(-1)
    sz8 = scales.reshape(-1, GS)
    zr8 = zeros.reshape(-1, GS)
    iota_t = jnp.arange(N_TOK, dtype=jnp.int32)
    perm = jnp.argsort(x_flat)
    xs = jnp.take(x_flat, perm)
    inv = jnp.zeros((N_TOK,), jnp.int32).at[perm].set(iota_t)
    ip = (inv % NSTEP) * TB + inv // NSTEP
    posr = (iota_t % TB) * NSTEP + iota_t // TB
    xsp = jnp.take(xs, posr)
    sgg, zgg = _sc_gather_sz(xsp, sz8, zr8)
    oh8 = (jnp.mod(xsp, 8)[:, None]
           == jnp.arange(8, dtype=jnp.int32)[None, :]).astype(jnp.float32)
    sel_col = (iota_t % TB) * RB + jnp.mod(xsp, RB)
    oh32 = (sel_col[:, None] == jnp.arange(TB * RB, dtype=jnp.int32)[None, :]
            ).astype(jnp.int8).reshape(NSTEP, TB, TB * RB)
    emat = (jnp.arange(K, dtype=jnp.int32)[None, :] // GS
            == jnp.arange(G, dtype=jnp.int32)[:, None]).astype(jnp.float32)
    outp = _tc_dequant(xs, qweights, sgg, zgg, oh8, oh32, emat)
    out = _sc_unpermute(outp.reshape(N_TOK, K), ip)
    return out.reshape(*shape, K)
